# Initial kernel scaffold; baseline (speedup 1.0000x reference)
import jax
import jax.numpy as jnp
from jax.experimental import pallas as pl


def kernel(x, edge_index, edge_attr, hidden_state, W1, b1, W2, b2, gru_kernel, gru_recurrent, gru_bias):
    raise NotImplementedError("write your pallas kernel here")


# speedup vs baseline: 1.2044x; 1.2044x over previous
import functools

import jax
import jax.numpy as jnp
from jax import lax
from jax.experimental import pallas as pl
from jax.experimental.pallas import tpu as pltpu
from jax.experimental.pallas import tpu_sc as plsc

N = 10000
E = 160000
H = 32
ED = 16

NUM_CORES = 2
NUM_SUBCORES = 16
ROWS_PER_SUBCORE = N // NUM_SUBCORES

GW = 128
SW = 128
EB = 256


def _vector_mesh():
    return plsc.VectorSubcoreMesh(core_axis_name="core", subcore_axis_name="subcore")


_SC_PARAMS = pltpu.CompilerParams(use_tc_tiling_on_sc=False)


def _gather_nf(x, col2d):
    @functools.partial(
        pl.kernel,
        out_type=jax.ShapeDtypeStruct((E, H), jnp.float32),
        mesh=_vector_mesh(),
        compiler_params=_SC_PARAMS,
    )
    def gather_kernel(x_hbm, i_hbm, o_hbm):
        def body(i_vmem, o_vmem):
            pltpu.sync_copy(x_hbm.at[i_vmem.at[0]], o_vmem)

        pltpu.emit_pipeline(
            body,
            grid=(E // GW,),
            in_specs=[pl.BlockSpec((1, GW), lambda i: (0, i))],
            out_specs=[pl.BlockSpec((GW, H), lambda i: (i, 0))],
            core_axis_name=("core", "subcore"),
            dimension_semantics=(pltpu.PARALLEL,),
        )(i_hbm, o_hbm)

    return gather_kernel(x, col2d)


def _edge_messages(edge_attr, nf, W1, b1, W2p, b2p):
    def body(ea_ref, nf_ref, w1_ref, b1_ref, w2_ref, b2_ref, out_ref):
        h1 = jnp.maximum(
            jnp.dot(ea_ref[...], w1_ref[...], preferred_element_type=jnp.float32)
            + b1_ref[...],
            0.0,
        )
        ew = (
            jnp.dot(h1, w2_ref[...], preferred_element_type=jnp.float32)
            + b2_ref[...]
        )
        nf = nf_ref[...]
        nf_rep = jnp.reshape(
            jnp.broadcast_to(nf[:, :, None], (EB, H, H)), (EB, H * H)
        )
        p = ew * nf_rep
        m = p[:, :512] + p[:, 512:]
        m = m[:, :256] + m[:, 256:]
        m = m[:, :128] + m[:, 128:]
        m = m[:, :64] + m[:, 64:]
        m = m[:, :32] + m[:, 32:]
        out_ref[...] = m

    return pl.pallas_call(
        body,
        grid=(E // EB,),
        in_specs=[
            pl.BlockSpec((EB, ED), lambda i: (i, 0)),
            pl.BlockSpec((EB, H), lambda i: (i, 0)),
            pl.BlockSpec((ED, H), lambda i: (0, 0)),
            pl.BlockSpec((1, H), lambda i: (0, 0)),
            pl.BlockSpec((H, H * H), lambda i: (0, 0)),
            pl.BlockSpec((1, H * H), lambda i: (0, 0)),
        ],
        out_specs=pl.BlockSpec((EB, H), lambda i: (i, 0)),
        out_shape=jax.ShapeDtypeStruct((E, H), jnp.float32),
    )(edge_attr, nf, W1, b1, W2p, b2p)


def _scatter_add(messages, row2d, zeros_slice):
    @functools.partial(
        pl.kernel,
        out_type=jax.ShapeDtypeStruct((NUM_CORES, N, H), jnp.float32),
        mesh=_vector_mesh(),
        scratch_types=[pltpu.VMEM_SHARED((N, H), jnp.float32)],
        compiler_params=_SC_PARAMS,
    )
    def scatter_kernel(m_hbm, i_hbm, z_hbm, o_hbm, acc):
        cid = lax.axis_index("core")
        sid = lax.axis_index("subcore")
        pltpu.sync_copy(
            z_hbm, acc.at[pl.ds(sid * ROWS_PER_SUBCORE, ROWS_PER_SUBCORE), :]
        )
        plsc.subcore_barrier()

        def body(m_vmem, i_vmem):
            pltpu.sync_copy(m_vmem, acc.at[i_vmem.at[0]], add=True)

        pltpu.emit_pipeline(
            body,
            grid=(E // SW,),
            in_specs=[
                pl.BlockSpec((SW, H), lambda i: (i, 0)),
                pl.BlockSpec((1, SW), lambda i: (0, i)),
            ],
            out_specs=[],
            core_axis_name=("core", "subcore"),
            dimension_semantics=(pltpu.PARALLEL,),
        )(m_hbm, i_hbm)
        plsc.subcore_barrier()
        pltpu.sync_copy(
            acc.at[pl.ds(sid * ROWS_PER_SUBCORE, ROWS_PER_SUBCORE), :],
            o_hbm.at[cid, pl.ds(sid * ROWS_PER_SUBCORE, ROWS_PER_SUBCORE), :],
        )

    return scatter_kernel(messages, row2d, zeros_slice)


def _gru(partial, hprev, gk, gr, gb0, gb1):
    def body(p_ref, h_ref, gk_ref, gr_ref, b0_ref, b1_ref, out_ref):
        agg = p_ref[0] + p_ref[1]
        hp = h_ref[...]
        gi = jnp.dot(agg, gk_ref[...], preferred_element_type=jnp.float32) + b0_ref[...]
        gh = jnp.dot(hp, gr_ref[...], preferred_element_type=jnp.float32) + b1_ref[...]
        z = jax.nn.sigmoid(gi[:, :H] + gh[:, :H])
        r = jax.nn.sigmoid(gi[:, H : 2 * H] + gh[:, H : 2 * H])
        hh = jnp.tanh(gi[:, 2 * H :] + r * gh[:, 2 * H :])
        out_ref[...] = z * hp + (1.0 - z) * hh

    return pl.pallas_call(
        body,
        out_shape=jax.ShapeDtypeStruct((N, H), jnp.float32),
    )(partial, hprev, gk, gr, gb0, gb1)


def kernel(x, edge_index, edge_attr, hidden_state, W1, b1, W2, b2,
           gru_kernel, gru_recurrent, gru_bias):
    row2d = edge_index[0].reshape(1, E)
    col2d = edge_index[1].reshape(1, E)
    W2p = W2.reshape(H, H, H).transpose(0, 2, 1).reshape(H, H * H)
    b2p = b2.reshape(H, H).T.reshape(1, H * H)
    zeros_slice = jnp.zeros((ROWS_PER_SUBCORE, H), jnp.float32)

    nf = _gather_nf(x, col2d)
    messages = _edge_messages(edge_attr, nf, W1, b1.reshape(1, H), W2p, b2p)
    partial = _scatter_add(messages, row2d, zeros_slice)
    hnew = _gru(partial, hidden_state[0], gru_kernel, gru_recurrent,
                gru_bias[0].reshape(1, 3 * H), gru_bias[1].reshape(1, 3 * H))
    return hnew, hnew[None, :, :]


# speedup vs baseline: 2.4612x; 2.0434x over previous
import functools

import jax
import jax.numpy as jnp
from jax import lax
from jax.experimental import pallas as pl
from jax.experimental.pallas import tpu as pltpu
from jax.experimental.pallas import tpu_sc as plsc

N = 10000
E = 160000
H = 32
ED = 16

NUM_CORES = 2
NUM_SUBCORES = 16
ROWS_PER_SUBCORE = N // NUM_SUBCORES

GW = 128
SW = 128
EB = 256


def _vector_mesh():
    return plsc.VectorSubcoreMesh(core_axis_name="core", subcore_axis_name="subcore")


_SC_PARAMS = pltpu.CompilerParams(use_tc_tiling_on_sc=False)


def _gather_nf(x, col2d):
    @functools.partial(
        pl.kernel,
        out_type=jax.ShapeDtypeStruct((E, H), jnp.float32),
        mesh=_vector_mesh(),
        compiler_params=_SC_PARAMS,
    )
    def gather_kernel(x_hbm, i_hbm, o_hbm):
        def body(i_vmem, o_vmem):
            pltpu.sync_copy(x_hbm.at[i_vmem.at[0]], o_vmem)

        pltpu.emit_pipeline(
            body,
            grid=(E // GW,),
            in_specs=[pl.BlockSpec((1, GW), lambda i: (0, i))],
            out_specs=[pl.BlockSpec((GW, H), lambda i: (i, 0))],
            core_axis_name=("core", "subcore"),
            dimension_semantics=(pltpu.PARALLEL,),
        )(i_hbm, o_hbm)

    return gather_kernel(x, col2d)


def _edge_messages(edge_attr, nf, W1, b1, W2p, b2p, Rrep):
    def body(ea_ref, nf_ref, w1_ref, b1_ref, w2_ref, b2_ref, r_ref, out_ref):
        h1 = jnp.maximum(
            jnp.dot(ea_ref[...], w1_ref[...], preferred_element_type=jnp.float32)
            + b1_ref[...],
            0.0,
        )
        ew = (
            jnp.dot(h1, w2_ref[...], preferred_element_type=jnp.float32)
            + b2_ref[...]
        )
        nf = nf_ref[...]
        nf_rep = jnp.dot(nf, r_ref[...], preferred_element_type=jnp.float32)
        p = ew * nf_rep
        m = p[:, :512] + p[:, 512:]
        m = m[:, :256] + m[:, 256:]
        m = m[:, :128] + m[:, 128:]
        m = m[:, :64] + m[:, 64:]
        m = m[:, :32] + m[:, 32:]
        out_ref[...] = m

    return pl.pallas_call(
        body,
        grid=(E // EB,),
        in_specs=[
            pl.BlockSpec((EB, ED), lambda i: (i, 0)),
            pl.BlockSpec((EB, H), lambda i: (i, 0)),
            pl.BlockSpec((ED, H), lambda i: (0, 0)),
            pl.BlockSpec((1, H), lambda i: (0, 0)),
            pl.BlockSpec((H, H * H), lambda i: (0, 0)),
            pl.BlockSpec((1, H * H), lambda i: (0, 0)),
            pl.BlockSpec((H, H * H), lambda i: (0, 0)),
        ],
        out_specs=pl.BlockSpec((EB, H), lambda i: (i, 0)),
        out_shape=jax.ShapeDtypeStruct((E, H), jnp.float32),
    )(edge_attr, nf, W1, b1, W2p, b2p, Rrep)


def _scatter_add(messages, row2d, zeros_slice):
    @functools.partial(
        pl.kernel,
        out_type=jax.ShapeDtypeStruct((NUM_CORES, N, H), jnp.float32),
        mesh=_vector_mesh(),
        scratch_types=[pltpu.VMEM_SHARED((N, H), jnp.float32)],
        compiler_params=_SC_PARAMS,
    )
    def scatter_kernel(m_hbm, i_hbm, z_hbm, o_hbm, acc):
        cid = lax.axis_index("core")
        sid = lax.axis_index("subcore")
        pltpu.sync_copy(
            z_hbm, acc.at[pl.ds(sid * ROWS_PER_SUBCORE, ROWS_PER_SUBCORE), :]
        )
        plsc.subcore_barrier()

        def body(m_vmem, i_vmem):
            pltpu.sync_copy(m_vmem, acc.at[i_vmem.at[0]], add=True)

        pltpu.emit_pipeline(
            body,
            grid=(E // SW,),
            in_specs=[
                pl.BlockSpec((SW, H), lambda i: (i, 0)),
                pl.BlockSpec((1, SW), lambda i: (0, i)),
            ],
            out_specs=[],
            core_axis_name=("core", "subcore"),
            dimension_semantics=(pltpu.PARALLEL,),
        )(m_hbm, i_hbm)
        plsc.subcore_barrier()
        pltpu.sync_copy(
            acc.at[pl.ds(sid * ROWS_PER_SUBCORE, ROWS_PER_SUBCORE), :],
            o_hbm.at[cid, pl.ds(sid * ROWS_PER_SUBCORE, ROWS_PER_SUBCORE), :],
        )

    return scatter_kernel(messages, row2d, zeros_slice)


def _gru(partial, hprev, gk, gr, gb0, gb1):
    def body(p_ref, h_ref, gk_ref, gr_ref, b0_ref, b1_ref, out_ref):
        agg = p_ref[0] + p_ref[1]
        hp = h_ref[...]
        gi = jnp.dot(agg, gk_ref[...], preferred_element_type=jnp.float32) + b0_ref[...]
        gh = jnp.dot(hp, gr_ref[...], preferred_element_type=jnp.float32) + b1_ref[...]
        z = jax.nn.sigmoid(gi[:, :H] + gh[:, :H])
        r = jax.nn.sigmoid(gi[:, H : 2 * H] + gh[:, H : 2 * H])
        hh = jnp.tanh(gi[:, 2 * H :] + r * gh[:, 2 * H :])
        out_ref[...] = z * hp + (1.0 - z) * hh

    return pl.pallas_call(
        body,
        out_shape=jax.ShapeDtypeStruct((N, H), jnp.float32),
    )(partial, hprev, gk, gr, gb0, gb1)


def kernel(x, edge_index, edge_attr, hidden_state, W1, b1, W2, b2,
           gru_kernel, gru_recurrent, gru_bias):
    row2d = edge_index[0].reshape(1, E)
    col2d = edge_index[1].reshape(1, E)
    W2p = W2.reshape(H, H, H).transpose(0, 2, 1).reshape(H, H * H)
    b2p = b2.reshape(H, H).T.reshape(1, H * H)
    zeros_slice = jnp.zeros((ROWS_PER_SUBCORE, H), jnp.float32)
    Rrep = jnp.repeat(jnp.eye(H, dtype=jnp.float32), H, axis=1)

    nf = _gather_nf(x, col2d)
    messages = _edge_messages(edge_attr, nf, W1, b1.reshape(1, H), W2p, b2p, Rrep)
    partial = _scatter_add(messages, row2d, zeros_slice)
    hnew = _gru(partial, hidden_state[0], gru_kernel, gru_recurrent,
                gru_bias[0].reshape(1, 3 * H), gru_bias[1].reshape(1, 3 * H))
    return hnew, hnew[None, :, :]
